# baseline (speedup 1.0000x reference)
import jax
import jax.numpy as jnp
from jax import lax
from jax.experimental import pallas as pl
from jax.experimental.pallas import tpu as pltpu
from jax.experimental.pallas import tpu_sc as plsc

_B, _C, _H, _W = 4, 384, 224, 224
_S = 16
_T = 56
_CG = 12


def _sc_body(x_hbm, o_hbm, sem):
    core = lax.axis_index("c")
    sub = lax.axis_index("s")
    wid = sub * 2 + core
    w16 = lax.rem(wid, _S)

    copies = []
    hi = lax.div(wid, _S)
    cb = w16 * _CG
    for t in range(2):
        b = hi * 2 + t
        copies.append(pltpu.make_async_copy(
            x_hbm.at[b, pl.ds(cb, _CG)],
            o_hbm.at[b, pl.ds(cb, _CG)],
            sem,
        ))
    for c in copies:
        c.start()

    @pl.when(wid < _S)
    def _():
        a = lax.div(wid, 4)
        b = lax.rem(wid, 4)
        cs = (_S + 4 * lax.div(wid, 4)) * _CG

        for av in range(4):
            @pl.when(a == av)
            def _(av=av):
                if av == 0:
                    cps = [pltpu.make_async_copy(
                        x_hbm.at[b, pl.ds(cs, _CG)],
                        o_hbm.at[b, pl.ds(cs, _CG)],
                        sem,
                    )]
                else:
                    k = _T * av
                    cps = [
                        pltpu.make_async_copy(
                            x_hbm.at[b, pl.ds(cs, _CG), pl.ds(k, _H - k)],
                            o_hbm.at[b, pl.ds(cs, _CG), pl.ds(0, _H - k)],
                            sem,
                        ),
                        pltpu.make_async_copy(
                            x_hbm.at[b, pl.ds(cs, _CG), pl.ds(0, k)],
                            o_hbm.at[b, pl.ds(cs, _CG), pl.ds(_H - k, k)],
                            sem,
                        ),
                    ]
                for c in cps:
                    c.start()
                for c in cps:
                    c.wait()

    for c in copies:
        c.wait()


def _tc_kernel(x_ref, y_ref, o_ref):
    del y_ref
    k = pl.program_id(1)
    for kv in range(12):
        @pl.when(k == kv)
        def _(kv=kv):
            sv = (kv // 3) * 4 + kv % 3 + 1
            a, r = sv // 4, sv % 4
            for j1 in range(4):
                q1 = (r + j1) % 4
                kk = (a + (r + j1) // 4) % 4
                lo, ql = j1 * _T, q1 * _T
                if kk == 0:
                    o_ref[0, :, :, lo:lo + _T] = x_ref[0, :, :, ql:ql + _T]
                else:
                    o_ref[0, :, : _H - _T * kk, lo:lo + _T] = (
                        x_ref[0, :, _T * kk:, ql:ql + _T])
                    o_ref[0, :, _H - _T * kk:, lo:lo + _T] = (
                        x_ref[0, :, : _T * kk, ql:ql + _T])


def _tc_group(b, k):
    return (b, _S + (k // 3) * 4 + k % 3 + 1, 0, 0)


def kernel(x):
    sc_kernel = pl.kernel(
        _sc_body,
        out_type=jax.ShapeDtypeStruct((_B, _C, _H, _W), x.dtype),
        mesh=plsc.VectorSubcoreMesh(core_axis_name="c", subcore_axis_name="s"),
        scratch_types=[pltpu.SemaphoreType.DMA],
    )
    y = sc_kernel(x)

    blk = (1, _CG, _H, _W)
    return pl.pallas_call(
        _tc_kernel,
        grid=(_B, 12),
        in_specs=[
            pl.BlockSpec(blk, _tc_group),
            pl.BlockSpec(memory_space=pltpu.MemorySpace.HBM),
        ],
        out_specs=pl.BlockSpec(blk, _tc_group),
        out_shape=jax.ShapeDtypeStruct((_B, _C, _H, _W), x.dtype),
        input_output_aliases={1: 0},
        compiler_params=pltpu.CompilerParams(
            dimension_semantics=("arbitrary", "arbitrary"),
        ),
    )(x, y)


# speedup vs baseline: 9.6541x; 9.6541x over previous
import jax
import jax.numpy as jnp
from jax.experimental import pallas as pl
from jax.experimental.pallas import tpu as pltpu

_B, _C, _H, _W = 4, 384, 224, 224
_S = 16
_T = 56
_G = 32
_CG = _C // _G


def _shift_kernel(x_ref, o_ref):
    g = pl.program_id(1)
    s = jnp.where(g >= _S, g - _S, 0)

    @pl.when(s == 0)
    def _():
        o_ref[...] = x_ref[...]

    for sv in range(1, _S):
        @pl.when(s == sv)
        def _(sv=sv):
            a, r = sv // 4, sv % 4
            for j1 in range(4):
                q1 = (r + j1) % 4
                k = (a + (r + j1) // 4) % 4
                lo, ql = j1 * _T, q1 * _T
                if k == 0:
                    o_ref[0, 0, :, :, lo:lo + _T] = x_ref[0, 0, :, :, ql:ql + _T]
                else:
                    o_ref[0, 0, :, : _H - _T * k, lo:lo + _T] = (
                        x_ref[0, 0, :, _T * k:, ql:ql + _T])
                    o_ref[0, 0, :, _H - _T * k:, lo:lo + _T] = (
                        x_ref[0, 0, :, : _T * k, ql:ql + _T])


def kernel(x):
    xr = x.reshape(_B, _G, _CG, _H, _W)
    spec = pl.BlockSpec(
        (1, 1, _CG, _H, _W), lambda b, g: (b, g, 0, 0, 0)
    )
    out = pl.pallas_call(
        _shift_kernel,
        grid=(_B, _G),
        in_specs=[spec],
        out_specs=spec,
        out_shape=jax.ShapeDtypeStruct((_B, _G, _CG, _H, _W), x.dtype),
        compiler_params=pltpu.CompilerParams(
            dimension_semantics=("arbitrary", "arbitrary"),
        ),
    )(xr)
    return out.reshape(_B, _C, _H, _W)
